# baseline (speedup 1.0000x reference)
import functools

import jax
import jax.numpy as jnp
from jax import lax
from jax.experimental import pallas as pl
from jax.experimental.pallas import tpu as pltpu
from jax.experimental.pallas import tpu_sc as plsc

NUM_BINS = 65536
N = 8388608

NC = 2
NS = 16
NW = NC * NS
L = 16

PER_W = N // NW
CHUNK = 16384
NCH = PER_W // CHUNK
ROWS = CHUNK // 128


def _hist_body(inds_hbm, parts_hbm, stage, hist, sem0, sem1):
    c = lax.axis_index("c")
    s = lax.axis_index("s")
    wid = s * NC + c
    rbase = wid * (PER_W // 128)

    sems = (sem0, sem1)

    def start(ch, buf):
        return pltpu.async_copy(
            inds_hbm.at[pl.ds(rbase + ch * ROWS, ROWS)],
            stage.at[buf],
            sems[buf],
        )

    def process(buf):
        @plsc.parallel_loop(0, ROWS, step=1, unroll=2)
        def _(r):
            for k in range(128 // L):
                idx = stage[buf, r, pl.ds(k * L, L)]
                cnt, last = plsc.scan_count(idx)
                hi = lax.shift_right_logical(idx, 7)
                lo = jnp.bitwise_and(idx, 127)
                plsc.addupdate_scatter(hist, [hi, lo], cnt, mask=last)

    start(0, 0)
    start(1, 1)

    @plsc.parallel_loop(0, NUM_BINS // 128, step=1, unroll=2)
    def _(r):
        for k in range(128 // L):
            hist[r, pl.ds(k * L, L)] = jnp.zeros((L,), jnp.int32)

    def wait(buf):
        pltpu.make_async_copy(
            inds_hbm.at[pl.ds(0, ROWS)], stage.at[buf], sems[buf]
        ).wait()

    def pair_body(p, carry):
        wait(0)
        process(0)
        start(2 * p + 2, 0)
        wait(1)
        process(1)
        start(2 * p + 3, 1)
        return carry

    lax.fori_loop(0, NCH // 2 - 1, pair_body, 0)
    wait(0)
    process(0)
    wait(1)
    process(1)

    pltpu.sync_copy(hist, parts_hbm.at[wid])


_hist_kernel = pl.kernel(
    _hist_body,
    out_type=jax.ShapeDtypeStruct((NW, NUM_BINS // 128, 128), jnp.int32),
    mesh=plsc.VectorSubcoreMesh(
        core_axis_name="c", subcore_axis_name="s", num_cores=NC,
        num_subcores=NS,
    ),
    scratch_types=[
        pltpu.VMEM((2, ROWS, 128), jnp.int32),
        pltpu.VMEM((NUM_BINS // 128, 128), jnp.int32),
        pltpu.SemaphoreType.DMA,
        pltpu.SemaphoreType.DMA,
    ],
    compiler_params=pltpu.CompilerParams(needs_layout_passes=False),
)


def _combine_body(parts_ref, counts_ref, s_ref, ss_ref):
    x = parts_ref[...]
    counts = jnp.sum(x, axis=0, keepdims=True)
    counts_ref[...] = counts
    b = lax.broadcasted_iota(jnp.int32, (1, NUM_BINS), 1)
    s_ref[...] = jnp.sum(counts * b).reshape(1, 1)
    ss_ref[...] = jnp.sum(counts * (b * b)).reshape(1, 1)


_combine_kernel = pl.pallas_call(
    _combine_body,
    out_shape=(
        jax.ShapeDtypeStruct((1, NUM_BINS), jnp.int32),
        jax.ShapeDtypeStruct((1, 1), jnp.int32),
        jax.ShapeDtypeStruct((1, 1), jnp.int32),
    ),
)


@jax.jit
def kernel(inds):
    parts = _hist_kernel(inds.reshape(N // 128, 128)).reshape(NW, NUM_BINS)
    counts2d, s2d, ss2d = _combine_kernel(parts)
    counts = counts2d.reshape(NUM_BINS)
    s = s2d[0, 0]
    ss = ss2d[0, 0]
    h_min = jnp.float32(0.0)
    h_max = jnp.float32(NUM_BINS - 1)
    num = jnp.int32(N)
    limits = jnp.arange(NUM_BINS + 1, dtype=jnp.int32)
    return (h_min, h_max, num, s, ss, limits, counts)


# speedup vs baseline: 1.1220x; 1.1220x over previous
import functools

import jax
import jax.numpy as jnp
from jax import lax
from jax.experimental import pallas as pl
from jax.experimental.pallas import tpu as pltpu
from jax.experimental.pallas import tpu_sc as plsc

NUM_BINS = 65536
N = 8388608

NC = 2
NS = 16
NW = NC * NS
L = 16

PER_W = N // NW
CHUNK = 16384
NCH = PER_W // CHUNK
ROWS = CHUNK // 128


def _hist_body(inds_hbm, parts_hbm, stage, hist, sem0, sem1):
    c = lax.axis_index("c")
    s = lax.axis_index("s")
    wid = s * NC + c
    rbase = wid * (PER_W // 128)

    sems = (sem0, sem1)

    def start(ch, buf):
        return pltpu.async_copy(
            inds_hbm.at[pl.ds(rbase + ch * ROWS, ROWS)],
            stage.at[buf],
            sems[buf],
        )

    def process(buf):
        @plsc.parallel_loop(0, ROWS, step=1, unroll=2)
        def _(r):
            for k in range(128 // L):
                idx = stage[buf, r, pl.ds(k * L, L)]
                cnt, last = plsc.scan_count(idx)
                plsc.addupdate_scatter(hist, [idx], cnt, mask=last)

    start(0, 0)
    start(1, 1)

    @plsc.parallel_loop(0, NUM_BINS, step=L, unroll=16)
    def _(i):
        hist[pl.ds(i, L)] = jnp.zeros((L,), jnp.int32)

    def wait(buf):
        pltpu.make_async_copy(
            inds_hbm.at[pl.ds(0, ROWS)], stage.at[buf], sems[buf]
        ).wait()

    def pair_body(p, carry):
        wait(0)
        process(0)
        start(2 * p + 2, 0)
        wait(1)
        process(1)
        start(2 * p + 3, 1)
        return carry

    lax.fori_loop(0, NCH // 2 - 1, pair_body, 0)
    wait(0)
    process(0)
    wait(1)
    process(1)

    pltpu.sync_copy(hist, parts_hbm.at[wid])


_hist_kernel = pl.kernel(
    _hist_body,
    out_type=jax.ShapeDtypeStruct((NW, NUM_BINS), jnp.int32),
    mesh=plsc.VectorSubcoreMesh(
        core_axis_name="c", subcore_axis_name="s", num_cores=NC,
        num_subcores=NS,
    ),
    scratch_types=[
        pltpu.VMEM((2, ROWS, 128), jnp.int32),
        pltpu.VMEM((NUM_BINS,), jnp.int32),
        pltpu.SemaphoreType.DMA,
        pltpu.SemaphoreType.DMA,
    ],
    compiler_params=pltpu.CompilerParams(needs_layout_passes=False),
)


NBLK = 8
BW = NUM_BINS // NBLK


def _combine_body(parts_ref, counts_ref, s_ref, ss_ref):
    g = pl.program_id(0)
    x = parts_ref[...]
    counts = jnp.sum(x, axis=0, keepdims=True)
    counts_ref[...] = counts
    b = lax.broadcasted_iota(jnp.int32, (1, BW), 1) + g * BW
    sp = jnp.sum(counts * b).reshape(1, 1)
    ssp = jnp.sum(counts * (b * b)).reshape(1, 1)

    @pl.when(g == 0)
    def _():
        s_ref[...] = jnp.zeros((1, 1), jnp.int32)
        ss_ref[...] = jnp.zeros((1, 1), jnp.int32)

    s_ref[...] += sp
    ss_ref[...] += ssp


_combine_kernel = pl.pallas_call(
    _combine_body,
    grid=(NBLK,),
    in_specs=[pl.BlockSpec((NW, BW), lambda g: (0, g))],
    out_specs=(
        pl.BlockSpec((1, BW), lambda g: (0, g)),
        pl.BlockSpec((1, 1), lambda g: (0, 0)),
        pl.BlockSpec((1, 1), lambda g: (0, 0)),
    ),
    out_shape=(
        jax.ShapeDtypeStruct((1, NUM_BINS), jnp.int32),
        jax.ShapeDtypeStruct((1, 1), jnp.int32),
        jax.ShapeDtypeStruct((1, 1), jnp.int32),
    ),
    compiler_params=pltpu.CompilerParams(
        dimension_semantics=("arbitrary",),
    ),
)


@jax.jit
def kernel(inds):
    parts = _hist_kernel(inds.reshape(N // 128, 128))
    counts2d, s2d, ss2d = _combine_kernel(parts)
    counts = counts2d.reshape(NUM_BINS)
    s = s2d[0, 0]
    ss = ss2d[0, 0]
    h_min = jnp.float32(0.0)
    h_max = jnp.float32(NUM_BINS - 1)
    num = jnp.int32(N)
    limits = jnp.arange(NUM_BINS + 1, dtype=jnp.int32)
    return (h_min, h_max, num, s, ss, limits, counts)


# speedup vs baseline: 1.1675x; 1.0406x over previous
import jax
import jax.numpy as jnp
from jax import lax
from jax.experimental import pallas as pl
from jax.experimental.pallas import tpu as pltpu
from jax.experimental.pallas import tpu_sc as plsc

NUM_BINS = 65536
N = 8388608

NC = 2
NS = 16
NW = NC * NS
L = 16

PER_W = N // NW
CHUNK = 16384
NCH = PER_W // CHUNK
ROWS = CHUNK // 128


def _hist_body(inds_hbm, parts_hbm, stage, hist, sem0, sem1):
    c = lax.axis_index("c")
    s = lax.axis_index("s")
    wid = s * NC + c
    rbase = wid * (PER_W // 128)

    sems = (sem0, sem1)

    def start(ch, buf):
        return pltpu.async_copy(
            inds_hbm.at[pl.ds(rbase + ch * ROWS, ROWS)],
            stage.at[buf],
            sems[buf],
        )

    def process(buf):
        @plsc.parallel_loop(0, ROWS, step=1, unroll=2)
        def _(r):
            for k in range(128 // L):
                idx = stage[buf, r, pl.ds(k * L, L)]
                cnt, last = plsc.scan_count(idx)
                plsc.addupdate_scatter(hist, [idx], cnt, mask=last)

    start(0, 0)
    start(1, 1)

    @plsc.parallel_loop(0, NUM_BINS, step=L, unroll=16)
    def _(i):
        hist[pl.ds(i, L)] = jnp.zeros((L,), jnp.int32)

    def wait(buf):
        pltpu.make_async_copy(
            inds_hbm.at[pl.ds(0, ROWS)], stage.at[buf], sems[buf]
        ).wait()

    def pair_body(p, carry):
        wait(0)
        process(0)
        start(2 * p + 2, 0)
        wait(1)
        process(1)
        start(2 * p + 3, 1)
        return carry

    lax.fori_loop(0, NCH // 2 - 1, pair_body, 0)
    wait(0)
    process(0)
    wait(1)
    process(1)

    pltpu.sync_copy(hist, parts_hbm.at[wid])


_hist_kernel = pl.kernel(
    _hist_body,
    out_type=jax.ShapeDtypeStruct((NW, NUM_BINS), jnp.int32),
    mesh=plsc.VectorSubcoreMesh(
        core_axis_name="c", subcore_axis_name="s", num_cores=NC,
        num_subcores=NS,
    ),
    scratch_types=[
        pltpu.VMEM((2, ROWS, 128), jnp.int32),
        pltpu.VMEM((NUM_BINS,), jnp.int32),
        pltpu.SemaphoreType.DMA,
        pltpu.SemaphoreType.DMA,
    ],
    compiler_params=pltpu.CompilerParams(needs_layout_passes=False),
)


def _combine_body(parts_ref, counts_ref, s_ref, ss_ref):
    x = parts_ref[...]
    counts = jnp.sum(x, axis=0, keepdims=True)
    counts_ref[...] = counts
    b = lax.broadcasted_iota(jnp.int32, (1, NUM_BINS), 1)
    s_ref[...] = jnp.sum(counts * b).reshape(1, 1)
    ss_ref[...] = jnp.sum(counts * (b * b)).reshape(1, 1)


_combine_kernel = pl.pallas_call(
    _combine_body,
    out_shape=(
        jax.ShapeDtypeStruct((1, NUM_BINS), jnp.int32),
        jax.ShapeDtypeStruct((1, 1), jnp.int32),
        jax.ShapeDtypeStruct((1, 1), jnp.int32),
    ),
)


@jax.jit
def kernel(inds):
    parts = _hist_kernel(inds.reshape(N // 128, 128))
    counts2d, s2d, ss2d = _combine_kernel(parts)
    counts = counts2d.reshape(NUM_BINS)
    s = s2d[0, 0]
    ss = ss2d[0, 0]
    h_min = jnp.float32(0.0)
    h_max = jnp.float32(NUM_BINS - 1)
    num = jnp.int32(N)
    limits = jnp.arange(NUM_BINS + 1, dtype=jnp.int32)
    return (h_min, h_max, num, s, ss, limits, counts)


# speedup vs baseline: 1.1691x; 1.0014x over previous
import jax
import jax.numpy as jnp
from jax import lax
from jax.experimental import pallas as pl
from jax.experimental.pallas import tpu as pltpu
from jax.experimental.pallas import tpu_sc as plsc

NUM_BINS = 65536
N = 8388608

NC = 2
NS = 16
NW = NC * NS
L = 16

PER_W = N // NW
CHUNK = 16384
NCH = PER_W // CHUNK
ROWS = CHUNK // 128


def _hist_body(inds_hbm, parts_hbm, stage, hist, sem0, sem1):
    c = lax.axis_index("c")
    s = lax.axis_index("s")
    wid = s * NC + c
    rbase = wid * (PER_W // 128)

    sems = (sem0, sem1)

    def start(ch, buf):
        return pltpu.async_copy(
            inds_hbm.at[pl.ds(rbase + ch * ROWS, ROWS)],
            stage.at[buf],
            sems[buf],
        )

    def process(buf):
        @plsc.parallel_loop(0, ROWS, step=1, unroll=1)
        def _(r):
            for k in range(128 // L):
                idx = stage[buf, r, pl.ds(k * L, L)]
                cnt, last = plsc.scan_count(idx)
                plsc.addupdate_scatter(hist, [idx], cnt, mask=last)

    start(0, 0)
    start(1, 1)

    @plsc.parallel_loop(0, NUM_BINS, step=L, unroll=16)
    def _(i):
        hist[pl.ds(i, L)] = jnp.zeros((L,), jnp.int32)

    def wait(buf):
        pltpu.make_async_copy(
            inds_hbm.at[pl.ds(0, ROWS)], stage.at[buf], sems[buf]
        ).wait()

    def pair_body(p, carry):
        wait(0)
        process(0)
        start(2 * p + 2, 0)
        wait(1)
        process(1)
        start(2 * p + 3, 1)
        return carry

    lax.fori_loop(0, NCH // 2 - 1, pair_body, 0)
    wait(0)
    process(0)
    wait(1)
    process(1)

    pltpu.sync_copy(hist, parts_hbm.at[wid])


_hist_kernel = pl.kernel(
    _hist_body,
    out_type=jax.ShapeDtypeStruct((NW, NUM_BINS), jnp.int32),
    mesh=plsc.VectorSubcoreMesh(
        core_axis_name="c", subcore_axis_name="s", num_cores=NC,
        num_subcores=NS,
    ),
    scratch_types=[
        pltpu.VMEM((2, ROWS, 128), jnp.int32),
        pltpu.VMEM((NUM_BINS,), jnp.int32),
        pltpu.SemaphoreType.DMA,
        pltpu.SemaphoreType.DMA,
    ],
    compiler_params=pltpu.CompilerParams(needs_layout_passes=False),
)


def _combine_body(parts_ref, counts_ref, s_ref, ss_ref):
    x = parts_ref[...]
    counts = jnp.sum(x, axis=0, keepdims=True)
    counts_ref[...] = counts
    b = lax.broadcasted_iota(jnp.int32, (1, NUM_BINS), 1)
    s_ref[...] = jnp.sum(counts * b).reshape(1, 1)
    ss_ref[...] = jnp.sum(counts * (b * b)).reshape(1, 1)


_combine_kernel = pl.pallas_call(
    _combine_body,
    out_shape=(
        jax.ShapeDtypeStruct((1, NUM_BINS), jnp.int32),
        jax.ShapeDtypeStruct((1, 1), jnp.int32),
        jax.ShapeDtypeStruct((1, 1), jnp.int32),
    ),
)


@jax.jit
def kernel(inds):
    parts = _hist_kernel(inds.reshape(N // 128, 128))
    counts2d, s2d, ss2d = _combine_kernel(parts)
    counts = counts2d.reshape(NUM_BINS)
    s = s2d[0, 0]
    ss = ss2d[0, 0]
    h_min = jnp.float32(0.0)
    h_max = jnp.float32(NUM_BINS - 1)
    num = jnp.int32(N)
    limits = jnp.arange(NUM_BINS + 1, dtype=jnp.int32)
    return (h_min, h_max, num, s, ss, limits, counts)
